# baseline (speedup 1.0000x reference)
import functools

import jax
import jax.numpy as jnp
from jax import lax
from jax.experimental import pallas as pl
from jax.experimental.pallas import tpu as pltpu
from jax.experimental.pallas import tpu_sc as plsc

SKU_D = 64
CAT_D = 32
EVT_D = 16
OUT_D = SKU_D + CAT_D + EVT_D
CAT_V = 1001
EVT_V = 11

NW = 32
BL = 128
L = 16
NDH = OUT_D // 8
SKU_DH = SKU_D // 8
CE_DH = NDH - SKU_DH


@functools.lru_cache(maxsize=None)
def _make_kernel(B: int, H: int):
    nbh = B // BL
    bh_per_w = nbh // NW
    assert nbh % NW == 0 and H % 2 == 0
    mesh = plsc.VectorSubcoreMesh(core_axis_name="c", subcore_axis_name="s")

    @functools.partial(
        pl.kernel,
        out_type=jax.ShapeDtypeStruct((H, NDH, nbh, 8, BL), jnp.float32),
        mesh=mesh,
        compiler_params=pltpu.CompilerParams(
            use_tc_tiling_on_sc=False, needs_layout_passes=False),
        scratch_types=[
            pltpu.VMEM((BL * H,), jnp.int32),
            pltpu.VMEM((BL * H,), jnp.int32),
            pltpu.VMEM((BL * H,), jnp.int32),
            pltpu.VMEM((BL,), jnp.int32),
            pltpu.VMEM((BL,), jnp.int32),
            pltpu.VMEM((BL, SKU_D), jnp.float32),
            pltpu.VMEM((BL, SKU_D), jnp.float32),
            pltpu.VMEM((SKU_D, BL), jnp.float32),
            pltpu.VMEM((SKU_D, BL), jnp.float32),
            pltpu.VMEM((CAT_D + EVT_D, BL), jnp.float32),
            pltpu.VMEM((CAT_D + EVT_D, BL), jnp.float32),
            pltpu.VMEM((CAT_V * CAT_D,), jnp.float32),
            pltpu.VMEM((EVT_V * EVT_D,), jnp.float32),
            pltpu.SemaphoreType.DMA,
            pltpu.SemaphoreType.DMA,
            pltpu.SemaphoreType.DMA,
            pltpu.SemaphoreType.DMA,
        ],
    )
    def k(sidx_h, cidx_h, eidx_h, sku_tab, cat_tab, evt_tab, out,
          sidx_v, cidx_v, eidx_v, gl0, gl1, sbuf0, sbuf1,
          sfrag0, sfrag1, cfrag0, cfrag1, cat_v, evt_v,
          gsem0, gsem1, osem0, osem1):
        gl = (gl0, gl1)
        sbuf = (sbuf0, sbuf1)
        sfrag = (sfrag0, sfrag1)
        cfrag = (cfrag0, cfrag1)
        gsem = (gsem0, gsem1)
        osem = (osem0, osem1)

        wid = lax.axis_index("s") * 2 + lax.axis_index("c")

        pltpu.sync_copy(cat_tab, cat_v)
        pltpu.sync_copy(evt_tab, evt_v)

        iota = lax.iota(jnp.int32, L)
        iotaH = iota * H
        rots = [(iota + d) & (L - 1) for d in range(L)]

        def build_glist(q, h):
            for g in range(BL // L):
                vec = plsc.load_gather(sidx_v, [iotaH + (g * L * H + h)])
                gl[q][pl.ds(g * L, L)] = vec

        def fire_gather(q):
            return pltpu.async_copy(sku_tab.at[gl[q]], sbuf[q], gsem[q])

        def drain_writes(q, bh):
            for dh in range(SKU_DH):
                pltpu.make_async_copy(
                    sfrag[q].at[pl.ds(dh * 8, 8)], out.at[0, dh, bh],
                    osem[q]).wait()
            for f in range(CE_DH):
                pltpu.make_async_copy(
                    cfrag[q].at[pl.ds(f * 8, 8)], out.at[0, SKU_DH + f, bh],
                    osem[q]).wait()

        @pl.loop(0, bh_per_w)
        def _(bh_i):
            bh = wid * bh_per_w + bh_i
            pltpu.sync_copy(sidx_h.at[pl.ds(bh * BL * H, BL * H)], sidx_v)
            pltpu.sync_copy(cidx_h.at[pl.ds(bh * BL * H, BL * H)], cidx_v)
            pltpu.sync_copy(eidx_h.at[pl.ds(bh * BL * H, BL * H)], eidx_v)

            build_glist(0, 0)
            fire_gather(0)

            @pl.loop(0, H, step=2)
            def _(h0):
                for p in range(2):
                    h = h0 + p

                    @pl.when(h < H - 1)
                    def _():
                        build_glist(1 - p, h + 1)
                        fire_gather(1 - p)

                    @pl.when(h0 >= 2)
                    def _():
                        drain_writes(p, bh)

                    pltpu.make_async_copy(
                        sku_tab.at[gl[p]], sbuf[p], gsem[p]).wait()

                    @pl.loop(0, BL // L, unroll=2)
                    def _(g):
                        lane_rows = iotaH + (g * L * H + h)
                        ci = plsc.load_gather(cidx_v, [lane_rows]) * CAT_D
                        ei = plsc.load_gather(eidx_v, [lane_rows]) * EVT_D
                        rows = iota + g * L
                        for f in range(CE_DH):
                            for dl in range(8):
                                col = f * 8 + dl
                                if col < CAT_D:
                                    vals = plsc.load_gather(cat_v, [ci + col])
                                else:
                                    vals = plsc.load_gather(
                                        evt_v, [ei + (col - CAT_D)])
                                cfrag[p][col, pl.ds(g * L, L)] = vals
                        for q in range(SKU_D // L):
                            for d in range(L):
                                c = rots[d] + (q * L)
                                vals = plsc.load_gather(sbuf[p], [rows, c])
                                plsc.store_scatter(sfrag[p], [c, rows], vals)

                    for dh in range(SKU_DH):
                        pltpu.async_copy(
                            sfrag[p].at[pl.ds(dh * 8, 8)], out.at[h, dh, bh],
                            osem[p])
                    for f in range(CE_DH):
                        pltpu.async_copy(
                            cfrag[p].at[pl.ds(f * 8, 8)],
                            out.at[h, SKU_DH + f, bh], osem[p])

            for p in range(2):
                drain_writes(p, bh)

    return k


def kernel(sku, category, event_type, sku_table, category_table, event_type_table):
    B, H = sku.shape
    sku_i = sku.reshape(B * H).astype(jnp.int32)
    cat_i = category.reshape(B * H).astype(jnp.int32)
    evt_i = event_type.reshape(B * H).astype(jnp.int32)
    out5 = _make_kernel(B, H)(
        sku_i, cat_i, evt_i,
        sku_table.astype(jnp.float32),
        category_table.reshape(CAT_V * CAT_D).astype(jnp.float32),
        event_type_table.reshape(EVT_V * EVT_D).astype(jnp.float32),
    )
    return jnp.transpose(out5, (2, 4, 0, 1, 3)).reshape(B, H, OUT_D)


# speedup vs baseline: 1.0309x; 1.0309x over previous
import functools

import jax
import jax.numpy as jnp
from jax import lax
from jax.experimental import pallas as pl
from jax.experimental.pallas import tpu as pltpu
from jax.experimental.pallas import tpu_sc as plsc

SKU_D = 64
CAT_D = 32
EVT_D = 16
OUT_D = SKU_D + CAT_D + EVT_D
CAT_V = 1001
EVT_V = 11

NW = 32
BL = 128
L = 16
NDH = OUT_D // 8
SKU_DH = SKU_D // 8
CE_DH = NDH - SKU_DH


@functools.lru_cache(maxsize=None)
def _make_kernel(B: int, H: int):
    nbh = B // BL
    bh_per_w = nbh // NW
    assert nbh % NW == 0 and H % 2 == 0
    mesh = plsc.VectorSubcoreMesh(core_axis_name="c", subcore_axis_name="s")

    @functools.partial(
        pl.kernel,
        out_type=jax.ShapeDtypeStruct((H, NDH, nbh, 8, BL), jnp.float32),
        mesh=mesh,
        compiler_params=pltpu.CompilerParams(
            use_tc_tiling_on_sc=False, needs_layout_passes=False),
        scratch_types=[
            pltpu.VMEM((BL * H,), jnp.int32),
            pltpu.VMEM((BL * H,), jnp.int32),
            pltpu.VMEM((BL * H,), jnp.int32),
            pltpu.VMEM((BL,), jnp.int32),
            pltpu.VMEM((BL,), jnp.int32),
            pltpu.VMEM((BL, SKU_D), jnp.float32),
            pltpu.VMEM((BL, SKU_D), jnp.float32),
            pltpu.VMEM((NDH, 8, BL), jnp.float32),
            pltpu.VMEM((NDH, 8, BL), jnp.float32),
            pltpu.VMEM((CAT_V * CAT_D,), jnp.float32),
            pltpu.VMEM((EVT_V * EVT_D,), jnp.float32),
            pltpu.SemaphoreType.DMA,
            pltpu.SemaphoreType.DMA,
            pltpu.SemaphoreType.DMA,
            pltpu.SemaphoreType.DMA,
        ],
    )
    def k(sidx_h, cidx_h, eidx_h, sku_tab, cat_tab, evt_tab, out,
          sidx_v, cidx_v, eidx_v, gl0, gl1, sbuf0, sbuf1,
          frag0, frag1, cat_v, evt_v,
          gsem0, gsem1, osem0, osem1):
        gl = (gl0, gl1)
        sbuf = (sbuf0, sbuf1)
        frag = (frag0, frag1)
        gsem = (gsem0, gsem1)
        osem = (osem0, osem1)

        wid = lax.axis_index("s") * 2 + lax.axis_index("c")

        pltpu.sync_copy(cat_tab, cat_v)
        pltpu.sync_copy(evt_tab, evt_v)

        iota = lax.iota(jnp.int32, L)
        iotaH = iota * H
        rots = [(iota + d) & (L - 1) for d in range(L)]

        def build_glist(q, h):
            for g in range(BL // L):
                vec = plsc.load_gather(sidx_v, [iotaH + (g * L * H + h)])
                gl[q][pl.ds(g * L, L)] = vec

        def fire_gather(q):
            return pltpu.async_copy(sku_tab.at[gl[q]], sbuf[q], gsem[q])

        def drain_writes(q, bh):
            pltpu.make_async_copy(frag[q], out.at[0, :, bh], osem[q]).wait()

        @pl.loop(0, bh_per_w)
        def _(bh_i):
            bh = wid * bh_per_w + bh_i
            pltpu.sync_copy(sidx_h.at[pl.ds(bh * BL * H, BL * H)], sidx_v)
            pltpu.sync_copy(cidx_h.at[pl.ds(bh * BL * H, BL * H)], cidx_v)
            pltpu.sync_copy(eidx_h.at[pl.ds(bh * BL * H, BL * H)], eidx_v)

            build_glist(0, 0)
            fire_gather(0)

            @pl.loop(0, H, step=2)
            def _(h0):
                for p in range(2):
                    h = h0 + p

                    @pl.when(h < H - 1)
                    def _():
                        build_glist(1 - p, h + 1)
                        fire_gather(1 - p)

                    @pl.when(h0 >= 2)
                    def _():
                        drain_writes(p, bh)

                    pltpu.make_async_copy(
                        sku_tab.at[gl[p]], sbuf[p], gsem[p]).wait()

                    @pl.loop(0, BL // L)
                    def _(g):
                        lane_rows = iotaH + (g * L * H + h)
                        ci = plsc.load_gather(cidx_v, [lane_rows]) * CAT_D
                        ei = plsc.load_gather(eidx_v, [lane_rows]) * EVT_D
                        rows = iota + g * L
                        for f in range(CE_DH):
                            for dl in range(8):
                                col = f * 8 + dl
                                if col < CAT_D:
                                    vals = plsc.load_gather(cat_v, [ci + col])
                                else:
                                    vals = plsc.load_gather(
                                        evt_v, [ei + (col - CAT_D)])
                                frag[p][SKU_DH + f, dl,
                                        pl.ds(g * L, L)] = vals
                        for q in range(SKU_D // L):
                            for d in range(L):
                                c = rots[d] + (q * L)
                                vals = plsc.load_gather(sbuf[p], [rows, c])
                                plsc.store_scatter(
                                    frag[p], [c >> 3, c & 7, rows], vals)

                    pltpu.async_copy(frag[p], out.at[h, :, bh], osem[p])

            for p in range(2):
                drain_writes(p, bh)

    return k


def kernel(sku, category, event_type, sku_table, category_table, event_type_table):
    B, H = sku.shape
    sku_i = sku.reshape(B * H).astype(jnp.int32)
    cat_i = category.reshape(B * H).astype(jnp.int32)
    evt_i = event_type.reshape(B * H).astype(jnp.int32)
    out5 = _make_kernel(B, H)(
        sku_i, cat_i, evt_i,
        sku_table.astype(jnp.float32),
        category_table.reshape(CAT_V * CAT_D).astype(jnp.float32),
        event_type_table.reshape(EVT_V * EVT_D).astype(jnp.float32),
    )
    return jnp.transpose(out5, (2, 4, 0, 1, 3)).reshape(B, H, OUT_D)
